# baseline (speedup 1.0000x reference)
import functools

import jax
import jax.numpy as jnp
from jax import lax
from jax.experimental import pallas as pl
from jax.experimental.pallas import tpu as pltpu
from jax.experimental.pallas import tpu_sc as plsc

BATCH = 16384
EMBED_DIM = 64
NUM_CORES = 2
NUM_SUBCORES = 16
NUM_WORKERS = NUM_CORES * NUM_SUBCORES
ROWS_PER_WORKER = BATCH // NUM_WORKERS
NUM_CHUNKS = 4
CHUNK = ROWS_PER_WORKER // NUM_CHUNKS
LANES = 16
GROUPS = ROWS_PER_WORKER // LANES

_mesh = plsc.VectorSubcoreMesh(core_axis_name="c", subcore_axis_name="s")


@functools.partial(
    pl.kernel,
    out_type=jax.ShapeDtypeStruct((NUM_WORKERS, ROWS_PER_WORKER), jnp.float32),
    mesh=_mesh,
    compiler_params=pltpu.CompilerParams(
        needs_layout_passes=False, use_tc_tiling_on_sc=False),
    scratch_types=[
        pltpu.VMEM((NUM_CHUNKS, CHUNK), jnp.int32),
        pltpu.VMEM((NUM_CHUNKS, CHUNK), jnp.int32),
        pltpu.VMEM((ROWS_PER_WORKER, EMBED_DIM), jnp.float32),
        pltpu.VMEM((ROWS_PER_WORKER, EMBED_DIM), jnp.float32),
        pltpu.VMEM((ROWS_PER_WORKER,), jnp.float32),
        pltpu.SemaphoreType.DMA,
        pltpu.SemaphoreType.DMA,
    ],
)
def _mf_kernel(uid_hbm, iid_hbm, ut_hbm, it_hbm, out_hbm,
               idx_u, idx_v, rows_u, rows_v, out_vals, sem_u, sem_v):
    wid = lax.axis_index("s") * NUM_CORES + lax.axis_index("c")

    pltpu.sync_copy(uid_hbm.at[wid], idx_u)
    pltpu.sync_copy(iid_hbm.at[wid], idx_v)

    copies = []
    for j in range(NUM_CHUNKS):
        copies.append(pltpu.async_copy(
            ut_hbm.at[idx_u.at[j]], rows_u.at[pl.ds(j * CHUNK, CHUNK)], sem_u))
        copies.append(pltpu.async_copy(
            it_hbm.at[idx_v.at[j]], rows_v.at[pl.ds(j * CHUNK, CHUNK)], sem_v))
    for c in copies:
        c.wait()

    def body(g, carry):
        base = g * LANES
        sums = jnp.zeros((LANES,), jnp.float32)
        for i in range(LANES):
            r = base + i
            s = rows_u[r, pl.ds(0, LANES)] * rows_v[r, pl.ds(0, LANES)]
            for c in range(1, EMBED_DIM // LANES):
                u = rows_u[r, pl.ds(c * LANES, LANES)]
                v = rows_v[r, pl.ds(c * LANES, LANES)]
                s = s + u * v
            lane_mask = jnp.arange(LANES, dtype=jnp.int32) == i
            sums = jnp.where(lane_mask, jnp.sum(s), sums)
        out_vals[pl.ds(base, LANES)] = sums
        return carry

    lax.fori_loop(0, GROUPS, body, 0)

    pltpu.sync_copy(out_vals, out_hbm.at[wid])


def kernel(user_ids, item_ids, user_table, item_table):
    uid = user_ids.astype(jnp.int32).reshape(NUM_WORKERS, NUM_CHUNKS, CHUNK)
    iid = item_ids.astype(jnp.int32).reshape(NUM_WORKERS, NUM_CHUNKS, CHUNK)
    out = _mf_kernel(uid, iid, user_table, item_table)
    return out.reshape(BATCH)


# speedup vs baseline: 2.3648x; 2.3648x over previous
import functools

import jax
import jax.numpy as jnp
from jax import lax
from jax.experimental import pallas as pl
from jax.experimental.pallas import tpu as pltpu
from jax.experimental.pallas import tpu_sc as plsc

NUM_USERS = 1000000
BATCH = 16384
EMBED_DIM = 64
SUBROWS = 8
NUM_TILES = NUM_USERS // SUBROWS
NUM_CORES = 2
NUM_SUBCORES = 16
NUM_WORKERS = NUM_CORES * NUM_SUBCORES
ROWS_PER_WORKER = BATCH // NUM_WORKERS
CHUNK = 32
NUM_CHUNKS = ROWS_PER_WORKER // CHUNK
LANES = 16

_mesh = plsc.VectorSubcoreMesh(core_axis_name="c", subcore_axis_name="s")


@functools.partial(
    pl.kernel,
    out_type=jax.ShapeDtypeStruct((NUM_WORKERS, ROWS_PER_WORKER), jnp.float32),
    mesh=_mesh,
    compiler_params=pltpu.CompilerParams(needs_layout_passes=False),
    scratch_types=[
        pltpu.VMEM((ROWS_PER_WORKER,), jnp.int32),
        pltpu.VMEM((ROWS_PER_WORKER,), jnp.int32),
        pltpu.VMEM((CHUNK, EMBED_DIM), jnp.float32),
        pltpu.VMEM((CHUNK, EMBED_DIM), jnp.float32),
        pltpu.VMEM((ROWS_PER_WORKER,), jnp.float32),
        pltpu.SemaphoreType.DMA,
        pltpu.SemaphoreType.DMA,
    ],
)
def _mf_kernel(uid_hbm, iid_hbm, ut_hbm, it_hbm, out_hbm,
               uid_v, iid_v, rows_u, rows_v, out_vals,
               sem_u, sem_v):
    wid = lax.axis_index("s") * NUM_CORES + lax.axis_index("c")

    pltpu.sync_copy(uid_hbm.at[wid], uid_v)
    pltpu.sync_copy(iid_hbm.at[wid], iid_v)

    def chunk_body(ch, carry):
        base = ch * CHUNK
        copies = []
        for g in range(CHUNK // LANES):
            uvec = uid_v[pl.ds(base + g * LANES, LANES)]
            ivec = iid_v[pl.ds(base + g * LANES, LANES)]
            for i in range(LANES):
                k = g * LANES + i
                u_id = uvec[i]
                i_id = ivec[i]
                copies.append(pltpu.async_copy(
                    ut_hbm.at[lax.shift_right_logical(u_id, 3),
                              u_id & (SUBROWS - 1)],
                    rows_u.at[k], sem_u))
                copies.append(pltpu.async_copy(
                    it_hbm.at[lax.shift_right_logical(i_id, 3),
                              i_id & (SUBROWS - 1)],
                    rows_v.at[k], sem_v))
        for c in copies:
            c.wait()
        for g in range(CHUNK // LANES):
            sums = jnp.zeros((LANES,), jnp.float32)
            for i in range(LANES):
                k = g * LANES + i
                s = rows_u[k, pl.ds(0, LANES)] * rows_v[k, pl.ds(0, LANES)]
                for c in range(1, EMBED_DIM // LANES):
                    u = rows_u[k, pl.ds(c * LANES, LANES)]
                    v = rows_v[k, pl.ds(c * LANES, LANES)]
                    s = s + u * v
                lane_mask = jnp.arange(LANES, dtype=jnp.int32) == i
                sums = jnp.where(lane_mask, jnp.sum(s), sums)
            out_vals[pl.ds(base + g * LANES, LANES)] = sums
        return carry

    lax.fori_loop(0, NUM_CHUNKS, chunk_body, 0)

    pltpu.sync_copy(out_vals, out_hbm.at[wid])


def kernel(user_ids, item_ids, user_table, item_table):
    uid = user_ids.astype(jnp.int32).reshape(NUM_WORKERS, ROWS_PER_WORKER)
    iid = item_ids.astype(jnp.int32).reshape(NUM_WORKERS, ROWS_PER_WORKER)
    ut3 = user_table.reshape(NUM_TILES, SUBROWS, EMBED_DIM)
    it3 = item_table.reshape(NUM_TILES, SUBROWS, EMBED_DIM)
    out = _mf_kernel(uid, iid, ut3, it3)
    return out.reshape(BATCH)
